# baseline (speedup 1.0000x reference)
import jax
import jax.numpy as jnp
from jax.experimental import pallas as pl
from jax.experimental.pallas import tpu as pltpu

MAX_POS = 8192
EMBED_DIM = 1024
SEQ_LEN = 8192

_CHUNKS = (512, 512, 1024, 2048, 4096)
_OFFS = tuple(sum(_CHUNKS[:i]) for i in range(len(_CHUNKS)))
_N = len(_CHUNKS)


def _dma_kernel(pe_hbm, out_hbm, buf, in_sems, out_sems):
    for i in range(_N):
        pltpu.make_async_copy(
            pe_hbm.at[pl.ds(_OFFS[i], _CHUNKS[i]), :],
            buf.at[pl.ds(_OFFS[i], _CHUNKS[i]), :],
            in_sems.at[i],
        ).start()
    for i in range(_N):
        pltpu.make_async_copy(
            pe_hbm.at[pl.ds(_OFFS[i], _CHUNKS[i]), :],
            buf.at[pl.ds(_OFFS[i], _CHUNKS[i]), :],
            in_sems.at[i],
        ).wait()
        pltpu.make_async_copy(
            buf.at[pl.ds(_OFFS[i], _CHUNKS[i]), :],
            out_hbm.at[pl.ds(_OFFS[i], _CHUNKS[i]), :],
            out_sems.at[i],
        ).start()
    for i in range(_N):
        pltpu.make_async_copy(
            buf.at[pl.ds(_OFFS[i], _CHUNKS[i]), :],
            out_hbm.at[pl.ds(_OFFS[i], _CHUNKS[i]), :],
            out_sems.at[i],
        ).wait()


def kernel(x, pe):
    out = pl.pallas_call(
        _dma_kernel,
        in_specs=[pl.BlockSpec(memory_space=pl.ANY)],
        out_specs=pl.BlockSpec(memory_space=pl.ANY),
        out_shape=jax.ShapeDtypeStruct((SEQ_LEN, EMBED_DIM), pe.dtype),
        scratch_shapes=[
            pltpu.VMEM((SEQ_LEN, EMBED_DIM), jnp.float32),
            pltpu.SemaphoreType.DMA((_N,)),
            pltpu.SemaphoreType.DMA((_N,)),
        ],
    )(pe)
    return out[None]


# speedup vs baseline: 1.0428x; 1.0428x over previous
import jax
import jax.numpy as jnp
from jax.experimental import pallas as pl
from jax.experimental.pallas import tpu as pltpu

MAX_POS = 8192
EMBED_DIM = 1024
SEQ_LEN = 8192

_N = 4
_CH = SEQ_LEN // _N


def _dma_kernel(pe_hbm, out_hbm, buf, in_sems, out_sems):
    for i in range(_N):
        pltpu.make_async_copy(
            pe_hbm.at[pl.ds(i * _CH, _CH), :], buf.at[i], in_sems.at[i]
        ).start()
    for i in range(_N):
        pltpu.make_async_copy(
            pe_hbm.at[pl.ds(i * _CH, _CH), :], buf.at[i], in_sems.at[i]
        ).wait()
        pltpu.make_async_copy(
            buf.at[i], out_hbm.at[pl.ds(i * _CH, _CH), :], out_sems.at[i]
        ).start()
    for i in range(_N):
        pltpu.make_async_copy(
            buf.at[i], out_hbm.at[pl.ds(i * _CH, _CH), :], out_sems.at[i]
        ).wait()


def kernel(x, pe):
    out = pl.pallas_call(
        _dma_kernel,
        in_specs=[pl.BlockSpec(memory_space=pl.ANY)],
        out_specs=pl.BlockSpec(memory_space=pl.ANY),
        out_shape=jax.ShapeDtypeStruct((SEQ_LEN, EMBED_DIM), pe.dtype),
        scratch_shapes=[
            pltpu.VMEM((_N, _CH, EMBED_DIM), jnp.float32),
            pltpu.SemaphoreType.DMA((_N,)),
            pltpu.SemaphoreType.DMA((_N,)),
        ],
    )(pe)
    return out[None]
